# baseline (speedup 1.0000x reference)
import jax
import jax.numpy as jnp
from jax import lax
from jax.experimental import pallas as pl
from jax.experimental.pallas import tpu as pltpu
from jax.experimental.pallas import tpu_sc as plsc

N = 10000
E = 320000
D_IN = 128
HID = 64
D = 64
HALF = 32
NS = 16
L = 16
R = 632
NSTR = NS * R
CH = 128
NCH = 160
GRP = 4
NGRP = NCH // GRP
ROWBYTES = CH * HALF * 4
EPT = NCH * CH
E_PAD = NS * EPT
SINK = N
ALPHA = 0.9
ROWB = 2000

SUBS = ((0, 128), (128, 128), (256, 128), (384, 128), (512, 120))


def _rsqrt16(d):
    xi = lax.bitcast_convert_type(d, jnp.int32)
    yi = jnp.int32(0x5F3759DF) - (xi >> 1)
    y = lax.bitcast_convert_type(yi, jnp.float32)
    for _ in range(3):
        y = y * (1.5 - 0.5 * d * y * y)
    return y


def _sc_body(src_hbm, dst_hbm, h_hbm, deg_hbm, out_hbm,
             q_sh, acc_sh,
             sidx0, didx0, sidx1, didx1, sidx2, didx2, sidx3, didx3,
             rows0, rows1, rows2, rows3,
             z_v,
             g_v, s_v, is_v,
             si0, si1, si2, si3, gs0, gs1, gs2, gs3, ss0, ss1, ss2, ss3):
    c = lax.axis_index("c")
    s = lax.axis_index("s")
    node_base = s * R
    h_base = c * NSTR + node_base
    gbase = s * NCH

    rows = (rows0, rows1, rows2, rows3)
    sidxs = (sidx0, sidx1, sidx2, sidx3)
    didxs = (didx0, didx1, didx2, didx3)
    isems = (si0, si1, si2, si3)
    gsems = (gs0, gs1, gs2, gs3)
    ssems = (ss0, ss1, ss2, ss3)

    def fetch_idx(g, slot):
        pltpu.async_copy(src_hbm.at[c, pl.ds(gbase + g * GRP, GRP)],
                         sidxs[slot], isems[slot])
        pltpu.async_copy(dst_hbm.at[pl.ds(gbase + g * GRP, GRP)],
                         didxs[slot], isems[slot])

    def drain_idx(slot):
        pltpu.make_async_copy(src_hbm.at[0, pl.ds(0, GRP)], sidxs[slot],
                              isems[slot]).wait()
        pltpu.make_async_copy(src_hbm.at[0, pl.ds(0, GRP)], didxs[slot],
                              isems[slot]).wait()

    def drain_scatter(b):
        pltpu.make_async_copy(h_hbm.at[pl.ds(0, CH)], rows[b], ssems[b]).wait()

    zeros16 = jnp.zeros((L,), jnp.float32)

    def init_row(i, _):
        z_v[i, pl.ds(0, L)] = zeros16
        z_v[i, pl.ds(L, L)] = zeros16
        return ()
    lax.fori_loop(0, CH, init_row, ())

    for off, sz in SUBS:
        pltpu.sync_copy(z_v.at[pl.ds(0, sz)], acc_sh.at[pl.ds(node_base + off, sz)])

    pltpu.sync_copy(deg_hbm.at[pl.ds(h_base, R)], s_v)

    def const_body(i, _):
        d = s_v[i, :] + 1.0
        is_v[i, :] = _rsqrt16(d)
        s_v[i, :] = ALPHA / d
        return ()
    lax.fori_loop(0, R, const_body, ())

    for off, sz in SUBS:
        pltpu.sync_copy(h_hbm.at[pl.ds(h_base + off, sz)], rows1.at[pl.ds(0, sz)])

        def h_body(i, _, off=off):
            isr = is_v[off + i, :]
            lo = rows1[i, pl.ds(0, L)] * isr
            hi = rows1[i, pl.ds(L, L)] * isr
            rows1[i, pl.ds(0, L)] = lo
            rows1[i, pl.ds(L, L)] = hi
            g_v[off + i, pl.ds(0, L)] = lo * (1.0 - ALPHA)
            g_v[off + i, pl.ds(L, L)] = hi * (1.0 - ALPHA)
            return ()
        lax.fori_loop(0, sz, h_body, ())
        pltpu.sync_copy(rows1.at[pl.ds(0, sz)], q_sh.at[pl.ds(h_base + off, sz)])
        pltpu.sync_copy(rows1.at[pl.ds(0, sz)], out_hbm.at[pl.ds(h_base + off, sz)])
    plsc.subcore_barrier()

    def iter_body(t, _):
        for b in range(GRP):
            pltpu.async_copy(q_sh.at[pl.ds(0, CH)], rows[b], ssems[b])
        fetch_idx(0, 0)
        fetch_idx(1, 1)
        fetch_idx(2, 2)

        def edge_body(k, _):
            for j in range(4):
                g = 4 * k + j
                drain_idx(j)
                ds = []
                for b in range(GRP):
                    drain_scatter(b)
                    qtab = out_hbm if b == GRP - 1 else q_sh
                    ds.append(pltpu.async_copy(qtab.at[sidxs[j].at[b]],
                                               rows[b], gsems[b]))
                fetch_idx(g + 3, (j + 3) % 4)
                for b in range(GRP):
                    ds[b].wait()
                    pltpu.async_copy(rows[b], acc_sh.at[didxs[j].at[b]],
                                     ssems[b], add=True)
            return ()
        lax.fori_loop(0, NGRP // 4, edge_body, ())
        for slot in range(3):
            drain_idx(slot)
        for b in range(GRP):
            drain_scatter(b)
        plsc.subcore_barrier()

        bufs = ((rows0, rows1, gs0, gs1), (rows2, rows3, gs2, gs3))
        pf = {}
        wq = {}
        wqh = {}
        wz = []

        def prefetch(i):
            off, sz = SUBS[i]
            av, qv, sa, sq = bufs[i % 2]
            pf[i] = (
                pltpu.async_copy(acc_sh.at[pl.ds(node_base + off, sz)],
                                 av.at[pl.ds(0, sz)], sa),
                pltpu.async_copy(q_sh.at[pl.ds(h_base + off, sz)],
                                 qv.at[pl.ds(0, sz)], sq))

        prefetch(0)
        for i, (off, sz) in enumerate(SUBS):
            av, qv, _, _ = bufs[i % 2]
            if i >= 1:
                wq[i - 1].wait()
                wqh[i - 1].wait()
            if i + 1 < len(SUBS):
                prefetch(i + 1)
            pf[i][0].wait()
            pf[i][1].wait()

            def comb_body(i2, _, off=off, av=av, qv=qv):
                sr = s_v[off + i2, :]
                av[i2, pl.ds(0, L)] = sr * (av[i2, pl.ds(0, L)] + qv[i2, pl.ds(0, L)]) + g_v[off + i2, pl.ds(0, L)]
                av[i2, pl.ds(L, L)] = sr * (av[i2, pl.ds(L, L)] + qv[i2, pl.ds(L, L)]) + g_v[off + i2, pl.ds(L, L)]
                return ()
            lax.fori_loop(0, sz, comb_body, ())
            wq[i] = pltpu.async_copy(av.at[pl.ds(0, sz)],
                                     q_sh.at[pl.ds(h_base + off, sz)], ss0)
            wqh[i] = pltpu.async_copy(av.at[pl.ds(0, sz)],
                                      out_hbm.at[pl.ds(h_base + off, sz)], ss2)
            wz.append(pltpu.async_copy(z_v.at[pl.ds(0, sz)],
                                       acc_sh.at[pl.ds(node_base + off, sz)], ss1))
        wq[len(SUBS) - 1].wait()
        wqh[len(SUBS) - 1].wait()
        for d in wz:
            d.wait()
        plsc.subcore_barrier()
        return ()
    lax.fori_loop(0, 10, iter_body, ())

    for off, sz in SUBS:
        pltpu.sync_copy(q_sh.at[pl.ds(h_base + off, sz)], rows0.at[pl.ds(0, sz)])

        def out_body(i, _, off=off):
            isr = is_v[off + i, :]
            rows0[i, pl.ds(0, L)] = rows0[i, pl.ds(0, L)] / isr
            rows0[i, pl.ds(L, L)] = rows0[i, pl.ds(L, L)] / isr
            return ()
        lax.fori_loop(0, sz, out_body, ())
        pltpu.sync_copy(rows0.at[pl.ds(0, sz)], out_hbm.at[pl.ds(h_base + off, sz)])


def _deg_body(dst_hbm, deg_out, deg_sh, didx_v, ones_v, z16_v, sem):
    c = lax.axis_index("c")
    s = lax.axis_index("s")
    node_base = s * R
    gbase = s * NCH
    ones16 = jnp.ones((L,), jnp.float32)
    zeros16 = jnp.zeros((L,), jnp.float32)

    def init_row(i, _):
        ones_v[i, :] = ones16
        z16_v[i, :] = zeros16
        return ()
    lax.fori_loop(0, CH, init_row, ())
    for off, sz in SUBS:
        pltpu.sync_copy(z16_v.at[pl.ds(0, sz)], deg_sh.at[pl.ds(node_base + off, sz)])
    plsc.subcore_barrier()

    def deg_body(g, _):
        pltpu.async_copy(dst_hbm.at[pl.ds(gbase + g * GRP, GRP)],
                         didx_v, sem).wait()
        for b in range(GRP):
            pltpu.sync_copy(ones_v, deg_sh.at[didx_v.at[b]], add=True)
        return ()
    lax.fori_loop(0, NGRP, deg_body, ())
    plsc.subcore_barrier()
    pltpu.sync_copy(deg_sh.at[pl.ds(node_base, R)],
                    deg_out.at[pl.ds(c * NSTR + node_base, R)])


_deg = pl.kernel(
    _deg_body,
    out_type=jax.ShapeDtypeStruct((2 * NSTR, L), jnp.float32),
    mesh=plsc.VectorSubcoreMesh(core_axis_name="c", subcore_axis_name="s"),
    compiler_params=pltpu.CompilerParams(use_tc_tiling_on_sc=False),
    scratch_types=[
        pltpu.VMEM_SHARED((NSTR, L), jnp.float32),
        pltpu.VMEM((GRP, CH), jnp.int32),
        pltpu.VMEM((CH, L), jnp.float32),
        pltpu.VMEM((CH, L), jnp.float32),
        pltpu.SemaphoreType.DMA,
    ],
)


_sc_prop = pl.kernel(
    _sc_body,
    out_type=jax.ShapeDtypeStruct((2 * NSTR, HALF), jnp.float32),
    mesh=plsc.VectorSubcoreMesh(core_axis_name="c", subcore_axis_name="s"),
    compiler_params=pltpu.CompilerParams(use_tc_tiling_on_sc=False),
    scratch_types=[
        pltpu.VMEM_SHARED((2 * NSTR, HALF), jnp.float32),
        pltpu.VMEM_SHARED((NSTR, HALF), jnp.float32),
        pltpu.VMEM((GRP, CH), jnp.int32),
        pltpu.VMEM((GRP, CH), jnp.int32),
        pltpu.VMEM((GRP, CH), jnp.int32),
        pltpu.VMEM((GRP, CH), jnp.int32),
        pltpu.VMEM((GRP, CH), jnp.int32),
        pltpu.VMEM((GRP, CH), jnp.int32),
        pltpu.VMEM((GRP, CH), jnp.int32),
        pltpu.VMEM((GRP, CH), jnp.int32),
        pltpu.VMEM((CH, HALF), jnp.float32),
        pltpu.VMEM((CH, HALF), jnp.float32),
        pltpu.VMEM((CH, HALF), jnp.float32),
        pltpu.VMEM((CH, HALF), jnp.float32),
        pltpu.VMEM((CH, HALF), jnp.float32),
        pltpu.VMEM((R, HALF), jnp.float32),
        pltpu.VMEM((R, L), jnp.float32),
        pltpu.VMEM((R, L), jnp.float32),
        pltpu.SemaphoreType.DMA,
        pltpu.SemaphoreType.DMA,
        pltpu.SemaphoreType.DMA,
        pltpu.SemaphoreType.DMA,
        pltpu.SemaphoreType.DMA,
        pltpu.SemaphoreType.DMA,
        pltpu.SemaphoreType.DMA,
        pltpu.SemaphoreType.DMA,
        pltpu.SemaphoreType.DMA,
        pltpu.SemaphoreType.DMA,
        pltpu.SemaphoreType.DMA,
        pltpu.SemaphoreType.DMA,
    ],
)


def _mlp_body(x_ref, w1_ref, b1_ref, w2_ref, b2_ref, o_ref):
    h1 = jnp.maximum(
        jnp.dot(x_ref[...], w1_ref[...], preferred_element_type=jnp.float32)
        + b1_ref[...], 0.0)
    o_ref[...] = jnp.maximum(
        jnp.dot(h1, w2_ref[...], preferred_element_type=jnp.float32)
        + b2_ref[...], 0.0)


_mlp = pl.pallas_call(
    _mlp_body,
    grid=(N // ROWB,),
    in_specs=[
        pl.BlockSpec((ROWB, D_IN), lambda i: (i, 0)),
        pl.BlockSpec((D_IN, HID), lambda i: (0, 0)),
        pl.BlockSpec((1, HID), lambda i: (0, 0)),
        pl.BlockSpec((HID, D), lambda i: (0, 0)),
        pl.BlockSpec((1, D), lambda i: (0, 0)),
    ],
    out_specs=pl.BlockSpec((ROWB, D), lambda i: (i, 0)),
    out_shape=jax.ShapeDtypeStruct((N, D), jnp.float32),
)


def _softmax_body(x_ref, o_ref):
    x = x_ref[...]
    m = jnp.max(x, axis=1, keepdims=True)
    e = jnp.exp(x - m)
    o_ref[...] = e / jnp.sum(e, axis=1, keepdims=True)


_softmax = pl.pallas_call(
    _softmax_body,
    grid=(N // ROWB,),
    in_specs=[pl.BlockSpec((ROWB, D), lambda i: (i, 0))],
    out_specs=pl.BlockSpec((ROWB, D), lambda i: (i, 0)),
    out_shape=jax.ShapeDtypeStruct((N, D), jnp.float32),
)


def kernel(features, edge_index, W1, b1, W2, b2):
    h = _mlp(features, W1, b1.reshape(1, HID), W2, b2.reshape(1, D))
    hp = jnp.pad(h, ((0, NSTR - N), (0, 0)))
    h2 = jnp.concatenate([hp[:, :HALF], hp[:, HALF:]], axis=0)
    src = edge_index[0].astype(jnp.int32)
    dst = edge_index[1].astype(jnp.int32)
    pad = jnp.full((E_PAD + 3 * GRP * CH - E,), SINK, jnp.int32)
    src1 = jnp.concatenate([src, pad]).reshape(NS * NCH + 3 * GRP, CH)
    dst1 = jnp.concatenate([dst, pad]).reshape(NS * NCH + 3 * GRP, CH)
    src3 = jnp.stack([src1, src1 + NSTR])
    deg2 = _deg(dst1)
    p2 = _sc_prop(src3, dst1, h2, deg2)
    p = jnp.concatenate([p2[:N], p2[NSTR:NSTR + N]], axis=1)
    return _softmax(p)


# speedup vs baseline: 1.3565x; 1.3565x over previous
import jax
import jax.numpy as jnp
from jax import lax
from jax.experimental import pallas as pl
from jax.experimental.pallas import tpu as pltpu
from jax.experimental.pallas import tpu_sc as plsc

N = 10000
E = 320000
D_IN = 128
HID = 64
D = 64
HALF = 32
NS = 16
L = 16
R = 632
NSTR = NS * R
CH = 128
NCH = 160
GRP = 4
NGRP = NCH // GRP
ROWBYTES = CH * HALF * 4
EPT = NCH * CH
E_PAD = NS * EPT
SINK = N
ALPHA = 0.9
ROWB = 2000

SUBS = ((0, 128), (128, 128), (256, 128), (384, 128), (512, 120))


def _rsqrt16(d):
    xi = lax.bitcast_convert_type(d, jnp.int32)
    yi = jnp.int32(0x5F3759DF) - (xi >> 1)
    y = lax.bitcast_convert_type(yi, jnp.float32)
    for _ in range(3):
        y = y * (1.5 - 0.5 * d * y * y)
    return y


def _sc_body(src_hbm, dst_hbm, h_hbm, deg_hbm, out_hbm,
             q_sh, acc_sh,
             sidx0, didx0, sidx1, didx1, sidx2, didx2, sidx3, didx3,
             rows0, rows1,
             z_v,
             g_v, s_v, is_v, a_v, q_v,
             si0, si1, si2, si3, gs0, gs1, gs2, gs3, ss0, ss1, ss2, ss3):
    c = lax.axis_index("c")
    s = lax.axis_index("s")
    node_base = s * R
    h_base = c * NSTR + node_base
    ebase = s * EPT

    rows = (rows0, rows1)
    sidxs = (sidx0, sidx1, sidx2, sidx3)
    didxs = (didx0, didx1, didx2, didx3)
    isems = (si0, si1, si2, si3)
    gsems = (gs0, gs1, gs2, gs3)
    ssems = (ss0, ss1, ss2, ss3)

    def fetch_idx(g, slot):
        pltpu.async_copy(src_hbm.at[pl.ds(ebase + g * GRP * CH, GRP * CH)],
                         sidxs[slot], isems[slot])
        pltpu.async_copy(dst_hbm.at[pl.ds(ebase + g * GRP * CH, GRP * CH)],
                         didxs[slot], isems[slot])

    def drain_idx(slot):
        pltpu.make_async_copy(src_hbm.at[pl.ds(0, GRP * CH)], sidxs[slot],
                              isems[slot]).wait()
        pltpu.make_async_copy(src_hbm.at[pl.ds(0, GRP * CH)], didxs[slot],
                              isems[slot]).wait()

    def drain_scatter(b):
        pltpu.make_async_copy(h_hbm.at[pl.ds(0, GRP * CH)], rows[b],
                              ssems[b]).wait()

    zeros16 = jnp.zeros((L,), jnp.float32)

    def init_row(i, _):
        z_v[i, pl.ds(0, L)] = zeros16
        z_v[i, pl.ds(L, L)] = zeros16
        return ()
    lax.fori_loop(0, CH, init_row, ())

    for off, sz in SUBS:
        pltpu.sync_copy(z_v.at[pl.ds(0, sz)], acc_sh.at[pl.ds(node_base + off, sz)])

    pltpu.sync_copy(deg_hbm.at[pl.ds(h_base, R)], s_v)

    def const_body(i, _):
        d = s_v[i, :] + 1.0
        is_v[i, :] = _rsqrt16(d)
        s_v[i, :] = ALPHA / d
        return ()
    lax.fori_loop(0, R, const_body, ())

    for off, sz in SUBS:
        pltpu.sync_copy(h_hbm.at[pl.ds(h_base + off, sz)], q_v.at[pl.ds(0, sz)])

        def h_body(i, _, off=off):
            isr = is_v[off + i, :]
            lo = q_v[i, pl.ds(0, L)] * isr
            hi = q_v[i, pl.ds(L, L)] * isr
            q_v[i, pl.ds(0, L)] = lo
            q_v[i, pl.ds(L, L)] = hi
            g_v[off + i, pl.ds(0, L)] = lo * (1.0 - ALPHA)
            g_v[off + i, pl.ds(L, L)] = hi * (1.0 - ALPHA)
            return ()
        lax.fori_loop(0, sz, h_body, ())
        pltpu.sync_copy(q_v.at[pl.ds(0, sz)], q_sh.at[pl.ds(node_base + off, sz)])
    plsc.subcore_barrier()

    def iter_body(t, _):
        for b in range(2):
            pltpu.async_copy(q_sh.at[pl.ds(0, GRP * CH)], rows[b], ssems[b])
        fetch_idx(0, 0)
        fetch_idx(1, 1)

        def edge_body(k, _):
            for j in range(4):
                g = 4 * k + j
                b = j % 2
                drain_idx(j)
                drain_scatter(b)
                d = pltpu.async_copy(q_sh.at[sidxs[j]], rows[b], gsems[b])
                fetch_idx(g + 2, (j + 2) % 4)
                d.wait()
                pltpu.async_copy(rows[b], acc_sh.at[didxs[j]],
                                 ssems[b], add=True)
            return ()
        lax.fori_loop(0, NGRP // 4, edge_body, ())
        for slot in range(2):
            drain_idx(slot)
        for b in range(2):
            drain_scatter(b)
        plsc.subcore_barrier()

        bufs = ((a_v, q_v, gs0, gs1), (rows0, rows1, gs2, gs3))
        pf = {}
        wq = {}
        wz = []

        def prefetch(i):
            off, sz = SUBS[i]
            av, qv, sa, sq = bufs[i % 2]
            pf[i] = (
                pltpu.async_copy(acc_sh.at[pl.ds(node_base + off, sz)],
                                 av.at[pl.ds(0, sz)], sa),
                pltpu.async_copy(q_sh.at[pl.ds(node_base + off, sz)],
                                 qv.at[pl.ds(0, sz)], sq))

        prefetch(0)
        for i, (off, sz) in enumerate(SUBS):
            av, qv, _, _ = bufs[i % 2]
            if i >= 1:
                wq[i - 1].wait()
            if i + 1 < len(SUBS):
                prefetch(i + 1)
            pf[i][0].wait()
            pf[i][1].wait()

            def comb_body(i2, _, off=off, av=av, qv=qv):
                sr = s_v[off + i2, :]
                av[i2, pl.ds(0, L)] = sr * (av[i2, pl.ds(0, L)] + qv[i2, pl.ds(0, L)]) + g_v[off + i2, pl.ds(0, L)]
                av[i2, pl.ds(L, L)] = sr * (av[i2, pl.ds(L, L)] + qv[i2, pl.ds(L, L)]) + g_v[off + i2, pl.ds(L, L)]
                return ()
            lax.fori_loop(0, sz, comb_body, ())
            wq[i] = pltpu.async_copy(av.at[pl.ds(0, sz)],
                                     q_sh.at[pl.ds(node_base + off, sz)], ss0)
            wz.append(pltpu.async_copy(z_v.at[pl.ds(0, sz)],
                                       acc_sh.at[pl.ds(node_base + off, sz)], ss1))
        wq[len(SUBS) - 1].wait()
        for d in wz:
            d.wait()
        plsc.subcore_barrier()
        return ()
    lax.fori_loop(0, 10, iter_body, ())

    for off, sz in SUBS:
        pltpu.sync_copy(q_sh.at[pl.ds(node_base + off, sz)], a_v.at[pl.ds(0, sz)])

        def out_body(i, _, off=off):
            isr = is_v[off + i, :]
            a_v[i, pl.ds(0, L)] = a_v[i, pl.ds(0, L)] / isr
            a_v[i, pl.ds(L, L)] = a_v[i, pl.ds(L, L)] / isr
            return ()
        lax.fori_loop(0, sz, out_body, ())
        pltpu.sync_copy(a_v.at[pl.ds(0, sz)], out_hbm.at[pl.ds(h_base + off, sz)])


def _deg_body(dst_hbm, deg_out, deg_sh, didx_v, ones_v, z16_v, sem):
    c = lax.axis_index("c")
    s = lax.axis_index("s")
    node_base = s * R
    ebase = s * EPT
    ones16 = jnp.ones((L,), jnp.float32)
    zeros16 = jnp.zeros((L,), jnp.float32)

    def init_row(i, _):
        ones_v[i, :] = ones16
        z16_v[i % CH, :] = zeros16
        return ()
    lax.fori_loop(0, GRP * CH, init_row, ())
    for off, sz in SUBS:
        pltpu.sync_copy(z16_v.at[pl.ds(0, sz)], deg_sh.at[pl.ds(node_base + off, sz)])
    plsc.subcore_barrier()

    def deg_body(g, _):
        pltpu.async_copy(dst_hbm.at[pl.ds(ebase + g * GRP * CH, GRP * CH)],
                         didx_v, sem).wait()
        pltpu.sync_copy(ones_v, deg_sh.at[didx_v], add=True)
        return ()
    lax.fori_loop(0, NGRP, deg_body, ())
    plsc.subcore_barrier()
    pltpu.sync_copy(deg_sh.at[pl.ds(node_base, R)],
                    deg_out.at[pl.ds(c * NSTR + node_base, R)])


_deg = pl.kernel(
    _deg_body,
    out_type=jax.ShapeDtypeStruct((2 * NSTR, L), jnp.float32),
    mesh=plsc.VectorSubcoreMesh(core_axis_name="c", subcore_axis_name="s"),
    compiler_params=pltpu.CompilerParams(use_tc_tiling_on_sc=False),
    scratch_types=[
        pltpu.VMEM_SHARED((NSTR, L), jnp.float32),
        pltpu.VMEM((GRP * CH,), jnp.int32),
        pltpu.VMEM((GRP * CH, L), jnp.float32),
        pltpu.VMEM((CH, L), jnp.float32),
        pltpu.SemaphoreType.DMA,
    ],
)


_sc_prop = pl.kernel(
    _sc_body,
    out_type=jax.ShapeDtypeStruct((2 * NSTR, HALF), jnp.float32),
    mesh=plsc.VectorSubcoreMesh(core_axis_name="c", subcore_axis_name="s"),
    compiler_params=pltpu.CompilerParams(use_tc_tiling_on_sc=False),
    scratch_types=[
        pltpu.VMEM_SHARED((NSTR, HALF), jnp.float32),
        pltpu.VMEM_SHARED((NSTR, HALF), jnp.float32),
        pltpu.VMEM((GRP * CH,), jnp.int32),
        pltpu.VMEM((GRP * CH,), jnp.int32),
        pltpu.VMEM((GRP * CH,), jnp.int32),
        pltpu.VMEM((GRP * CH,), jnp.int32),
        pltpu.VMEM((GRP * CH,), jnp.int32),
        pltpu.VMEM((GRP * CH,), jnp.int32),
        pltpu.VMEM((GRP * CH,), jnp.int32),
        pltpu.VMEM((GRP * CH,), jnp.int32),
        pltpu.VMEM((GRP * CH, HALF), jnp.float32),
        pltpu.VMEM((GRP * CH, HALF), jnp.float32),
        pltpu.VMEM((CH, HALF), jnp.float32),
        pltpu.VMEM((R, HALF), jnp.float32),
        pltpu.VMEM((R, L), jnp.float32),
        pltpu.VMEM((R, L), jnp.float32),
        pltpu.VMEM((CH, HALF), jnp.float32),
        pltpu.VMEM((CH, HALF), jnp.float32),
        pltpu.SemaphoreType.DMA,
        pltpu.SemaphoreType.DMA,
        pltpu.SemaphoreType.DMA,
        pltpu.SemaphoreType.DMA,
        pltpu.SemaphoreType.DMA,
        pltpu.SemaphoreType.DMA,
        pltpu.SemaphoreType.DMA,
        pltpu.SemaphoreType.DMA,
        pltpu.SemaphoreType.DMA,
        pltpu.SemaphoreType.DMA,
        pltpu.SemaphoreType.DMA,
        pltpu.SemaphoreType.DMA,
    ],
)


def _mlp_body(x_ref, w1_ref, b1_ref, w2_ref, b2_ref, o_ref):
    h1 = jnp.maximum(
        jnp.dot(x_ref[...], w1_ref[...], preferred_element_type=jnp.float32)
        + b1_ref[...], 0.0)
    o_ref[...] = jnp.maximum(
        jnp.dot(h1, w2_ref[...], preferred_element_type=jnp.float32)
        + b2_ref[...], 0.0)


_mlp = pl.pallas_call(
    _mlp_body,
    grid=(N // ROWB,),
    in_specs=[
        pl.BlockSpec((ROWB, D_IN), lambda i: (i, 0)),
        pl.BlockSpec((D_IN, HID), lambda i: (0, 0)),
        pl.BlockSpec((1, HID), lambda i: (0, 0)),
        pl.BlockSpec((HID, D), lambda i: (0, 0)),
        pl.BlockSpec((1, D), lambda i: (0, 0)),
    ],
    out_specs=pl.BlockSpec((ROWB, D), lambda i: (i, 0)),
    out_shape=jax.ShapeDtypeStruct((N, D), jnp.float32),
)


def _softmax_body(x_ref, o_ref):
    x = x_ref[...]
    m = jnp.max(x, axis=1, keepdims=True)
    e = jnp.exp(x - m)
    o_ref[...] = e / jnp.sum(e, axis=1, keepdims=True)


_softmax = pl.pallas_call(
    _softmax_body,
    grid=(N // ROWB,),
    in_specs=[pl.BlockSpec((ROWB, D), lambda i: (i, 0))],
    out_specs=pl.BlockSpec((ROWB, D), lambda i: (i, 0)),
    out_shape=jax.ShapeDtypeStruct((N, D), jnp.float32),
)


def kernel(features, edge_index, W1, b1, W2, b2):
    h = _mlp(features, W1, b1.reshape(1, HID), W2, b2.reshape(1, D))
    hp = jnp.pad(h, ((0, NSTR - N), (0, 0)))
    h2 = jnp.concatenate([hp[:, :HALF], hp[:, HALF:]], axis=0)
    src = edge_index[0].astype(jnp.int32)
    dst = edge_index[1].astype(jnp.int32)
    pad = jnp.full((E_PAD + 3 * GRP * CH - E,), SINK, jnp.int32)
    src1 = jnp.concatenate([src, pad])
    dst1 = jnp.concatenate([dst, pad])
    deg2 = _deg(dst1)
    p2 = _sc_prop(src1, dst1, h2, deg2)
    p = jnp.concatenate([p2[:N], p2[NSTR:NSTR + N]], axis=1)
    return _softmax(p)
